# baseline (speedup 1.0000x reference)
import jax
import jax.numpy as jnp
from jax import lax
from jax.experimental import pallas as pl
from jax.experimental.pallas import tpu as pltpu
from jax.experimental.pallas import tpu_sc as plsc

BATCH = 4096
HIST = 50
EMBED = 64
NP = 100000
B = BATCH * HIST
ROW = 128
NROWS = B // ROW
ROWS_PER_H = BATCH // ROW
NUM_CORES = 2
NUM_SUBCORES = 16
NUM_WORKERS = NUM_CORES * NUM_SUBCORES
ROWS_PER_W = NROWS // NUM_WORKERS
TPAD = 2 * EMBED
PITCH = ROW + 1
RETILE_BLK = 256
RETILE_GRID = 200
SPLIT = 191 * RETILE_BLK
RETILE_ROWS = RETILE_GRID * RETILE_BLK


def _retile_tc(ta_ref, tb_ref, out_ref):
    out_ref[...] = jnp.concatenate([ta_ref[...].T, tb_ref[...].T], axis=1)


def _gather_body(table_hbm, idx_hbm, out_hbm,
                 idx_all, rows0, rows1, tb0, tb1,
                 gsem0, gsem1, wsem0, wsem1):
    wid = lax.axis_index("s") * NUM_CORES + lax.axis_index("c")
    r0 = wid * ROWS_PER_W
    bufs = ((rows0, tb0, gsem0, wsem0), (rows1, tb1, gsem1, wsem1))

    pltpu.sync_copy(idx_hbm.at[pl.ds(r0, ROWS_PER_W)], idx_all)

    def stage(i_local, b):
        rows_v, _, gsem, _ = bufs[b]
        pltpu.async_copy(table_hbm.at[idx_all.at[i_local]], rows_v, gsem)

    iota = lax.iota(jnp.int32, 16)
    dvecs = tuple(iota + 16 * k for k in range(EMBED // 16))

    def wb_descs(i_local, b):
        _, tbuf, _, wsem = bufs[b]
        r = r0 + i_local
        h = r // ROWS_PER_H
        c = r % ROWS_PER_H
        return [(tbuf.at[pl.ds(8 * t, 8), pl.ds(0, ROW)],
                 out_hbm.at[h, t, c], wsem) for t in range(EMBED // 8)]

    def drain_wb(i_local, b):
        for src, dst, wsem in wb_descs(i_local, b):
            pltpu.make_async_copy(src, dst, wsem).wait()

    def finish(i_local, b):
        rows_v, tbuf, gsem, wsem = bufs[b]
        pltpu.make_async_copy(
            table_hbm.at[idx_all.at[i_local]], rows_v, gsem).wait()

        @pl.when(i_local >= 2)
        def _():
            drain_wb(i_local, b)

        @plsc.parallel_loop(0, ROW, step=1, unroll=4)
        def _(l):
            lsplat = jnp.full((16,), l, jnp.int32)
            for k in range(EMBED // 16):
                v = rows_v[l, pl.ds(16 * k, 16)]
                plsc.store_scatter(tbuf, [dvecs[k], lsplat], v)

        for src, dst, wsem_ in wb_descs(i_local, b):
            pltpu.async_copy(src, dst, wsem_)

    stage(0, 0)

    def body(i, carry):
        il = 2 * i
        stage(il + 1, 1)
        finish(il, 0)

        @pl.when(i < ROWS_PER_W // 2 - 1)
        def _():
            stage(il + 2, 0)

        finish(il + 1, 1)
        return carry

    lax.fori_loop(0, ROWS_PER_W // 2, body, 0)
    drain_wb(ROWS_PER_W - 2, 0)
    drain_wb(ROWS_PER_W - 1, 1)


@jax.jit
def kernel(x, table):
    xi = x.T.reshape(NROWS, ROW).astype(jnp.int32)
    idxT = jnp.where(xi < SPLIT, 2 * xi, 2 * (xi - SPLIT) + 1)
    tpair = pl.pallas_call(
        _retile_tc,
        grid=(RETILE_GRID,),
        in_specs=[
            pl.BlockSpec((EMBED, RETILE_BLK), lambda c: (0, c)),
            pl.BlockSpec((EMBED, RETILE_BLK), lambda c: (0, c + 191)),
        ],
        out_specs=pl.BlockSpec((RETILE_BLK, TPAD), lambda c: (c, 0)),
        out_shape=jax.ShapeDtypeStruct((RETILE_ROWS, TPAD), jnp.float32),
    )(table.T, table.T)
    tp = tpair.reshape(2 * RETILE_ROWS, EMBED)
    mesh = plsc.VectorSubcoreMesh(core_axis_name="c", subcore_axis_name="s")
    out5d = pl.kernel(
        _gather_body,
        mesh=mesh,
        out_type=jax.ShapeDtypeStruct(
            (HIST, EMBED // 8, ROWS_PER_H, 8, ROW), jnp.float32),
        scratch_types=[
            pltpu.VMEM((ROWS_PER_W, ROW), jnp.int32),
            pltpu.VMEM((ROW, EMBED), jnp.float32),
            pltpu.VMEM((ROW, EMBED), jnp.float32),
            pltpu.VMEM((EMBED, PITCH), jnp.float32),
            pltpu.VMEM((EMBED, PITCH), jnp.float32),
            pltpu.SemaphoreType.DMA,
            pltpu.SemaphoreType.DMA,
            pltpu.SemaphoreType.DMA,
            pltpu.SemaphoreType.DMA,
        ],
        compiler_params=pltpu.CompilerParams(
            use_tc_tiling_on_sc=False, needs_layout_passes=False),
    )(tp, idxT)
    return out5d.transpose(2, 4, 0, 1, 3).reshape(BATCH, HIST, EMBED)


# speedup vs baseline: 1.5920x; 1.5920x over previous
import jax
import jax.numpy as jnp
from jax import lax
from jax.experimental import pallas as pl
from jax.experimental.pallas import tpu as pltpu
from jax.experimental.pallas import tpu_sc as plsc

BATCH = 4096
HIST = 50
EMBED = 64
NP = 100000
B = BATCH * HIST
ROW = 128
NROWS = B // ROW
ROWS_PER_H = BATCH // ROW
NUM_CORES = 2
NUM_SUBCORES = 16
NUM_WORKERS = NUM_CORES * NUM_SUBCORES
ROWS_PER_W = NROWS // NUM_WORKERS
TPAD = 2 * EMBED
PITCH = ROW + 1


def _gather_body(table_hbm, idx_hbm, out_hbm,
                 idx_all, rows0, rows1, tb0, tb1,
                 gsem0, gsem1, wsem0, wsem1):
    wid = lax.axis_index("s") * NUM_CORES + lax.axis_index("c")
    r0 = wid * ROWS_PER_W
    bufs = ((rows0, tb0, gsem0, wsem0), (rows1, tb1, gsem1, wsem1))

    pltpu.sync_copy(idx_hbm.at[pl.ds(r0, ROWS_PER_W)], idx_all)

    def stage(i_local, b):
        rows_v, _, gsem, _ = bufs[b]
        pltpu.async_copy(table_hbm.at[idx_all.at[i_local]], rows_v, gsem)

    iota = lax.iota(jnp.int32, 16)
    dvecs = tuple(iota + 16 * k for k in range(EMBED // 16))

    def wb_descs(i_local, b):
        _, tbuf, _, wsem = bufs[b]
        r = r0 + i_local
        h = r // ROWS_PER_H
        c = r % ROWS_PER_H
        return [(tbuf.at[pl.ds(8 * t, 8), pl.ds(0, ROW)],
                 out_hbm.at[h, t, c], wsem) for t in range(EMBED // 8)]

    def drain_wb(i_local, b):
        for src, dst, wsem in wb_descs(i_local, b):
            pltpu.make_async_copy(src, dst, wsem).wait()

    def finish(i_local, b):
        rows_v, tbuf, gsem, wsem = bufs[b]
        pltpu.make_async_copy(
            table_hbm.at[idx_all.at[i_local]], rows_v, gsem).wait()

        @pl.when(i_local >= 2)
        def _():
            drain_wb(i_local, b)

        @plsc.parallel_loop(0, ROW, step=1, unroll=4)
        def _(l):
            lsplat = jnp.full((16,), l, jnp.int32)
            for k in range(EMBED // 16):
                v = rows_v[l, pl.ds(16 * k, 16)]
                plsc.store_scatter(tbuf, [dvecs[k], lsplat], v)

        for src, dst, wsem_ in wb_descs(i_local, b):
            pltpu.async_copy(src, dst, wsem_)

    stage(0, 0)

    def body(i, carry):
        il = 2 * i
        stage(il + 1, 1)
        finish(il, 0)

        @pl.when(i < ROWS_PER_W // 2 - 1)
        def _():
            stage(il + 2, 0)

        finish(il + 1, 1)
        return carry

    lax.fori_loop(0, ROWS_PER_W // 2, body, 0)
    drain_wb(ROWS_PER_W - 2, 0)
    drain_wb(ROWS_PER_W - 1, 1)


@jax.jit
def kernel(x, table):
    idxT = (x.T.reshape(NROWS, ROW) * 2).astype(jnp.int32)
    tp = jnp.pad(table, ((0, 0), (0, TPAD - EMBED))).reshape(2 * NP, EMBED)
    mesh = plsc.VectorSubcoreMesh(core_axis_name="c", subcore_axis_name="s")
    out5d = pl.kernel(
        _gather_body,
        mesh=mesh,
        out_type=jax.ShapeDtypeStruct(
            (HIST, EMBED // 8, ROWS_PER_H, 8, ROW), jnp.float32),
        scratch_types=[
            pltpu.VMEM((ROWS_PER_W, ROW), jnp.int32),
            pltpu.VMEM((ROW, EMBED), jnp.float32),
            pltpu.VMEM((ROW, EMBED), jnp.float32),
            pltpu.VMEM((EMBED, PITCH), jnp.float32),
            pltpu.VMEM((EMBED, PITCH), jnp.float32),
            pltpu.SemaphoreType.DMA,
            pltpu.SemaphoreType.DMA,
            pltpu.SemaphoreType.DMA,
            pltpu.SemaphoreType.DMA,
        ],
        compiler_params=pltpu.CompilerParams(
            use_tc_tiling_on_sc=False, needs_layout_passes=False),
    )(tp, idxT)
    return out5d.transpose(2, 4, 0, 1, 3).reshape(BATCH, HIST, EMBED)
